# baseline (speedup 1.0000x reference)
import functools

import jax
import jax.numpy as jnp
from jax import lax
from jax.experimental import pallas as pl
from jax.experimental.pallas import tpu as pltpu
from jax.experimental.pallas import tpu_sc as plsc

B, N, V_SIDE = 4, 16384, 32
V = V_SIDE ** 3
C = 256
PTS = B * N
PBLK = 2048

NC, NS, L = 2, 16, 16
NW = NC * NS
CG = C // (NW // B)
G = 2
NPASS = CG // G


def _mlp_body(xt_ref, w0_ref, b0_ref, w1_ref, b1_ref, w2_ref, b2_ref,
              ht_ref, flat_ref):
    xt = xt_ref[...]
    h = jnp.maximum(
        jnp.dot(w0_ref[...], xt, preferred_element_type=jnp.float32)
        + b0_ref[...], 0.0)
    h = jnp.maximum(
        jnp.dot(w1_ref[...], h, preferred_element_type=jnp.float32)
        + b1_ref[...], 0.0)
    h = jnp.maximum(
        jnp.dot(w2_ref[...], h, preferred_element_type=jnp.float32)
        + b2_ref[...], 0.0)
    ht_ref[...] = h
    vox = jnp.floor(xt * 32.0).astype(jnp.int32)
    flat = vox[0:1] * 1024 + vox[1:2] * 32 + vox[2:3]
    flat_ref[...] = flat.reshape(1, 1, PBLK)


_mlp_call = pl.pallas_call(
    _mlp_body,
    grid=(PTS // PBLK,),
    in_specs=[
        pl.BlockSpec((4, PBLK), lambda i: (0, i)),
        pl.BlockSpec((64, 4), lambda i: (0, 0)),
        pl.BlockSpec((64, 1), lambda i: (0, 0)),
        pl.BlockSpec((128, 64), lambda i: (0, 0)),
        pl.BlockSpec((128, 1), lambda i: (0, 0)),
        pl.BlockSpec((256, 128), lambda i: (0, 0)),
        pl.BlockSpec((256, 1), lambda i: (0, 0)),
    ],
    out_specs=[
        pl.BlockSpec((C, PBLK), lambda i: (0, i)),
        pl.BlockSpec((1, 1, PBLK), lambda i: (i, 0, 0)),
    ],
    out_shape=[
        jax.ShapeDtypeStruct((C, PTS), jnp.float32),
        jax.ShapeDtypeStruct((PTS // PBLK, 1, PBLK), jnp.int32),
    ],
)


def _sc_body(ht_hbm, flat_hbm, out_hbm, idx_v, h_v, grid_v, out_v):
    w = lax.axis_index("s") * NC + lax.axis_index("c")
    b = w // (NW // B)
    cg = w % (NW // B)
    pltpu.sync_copy(flat_hbm.at[b], idx_v)

    def zbody(i, carry):
        grid_v[pl.ds(i * L, L)] = jnp.zeros((L,), jnp.float32)
        return carry

    lax.fori_loop(0, G * V // L, zbody, 0)

    def pass_body(p, carry):
        c0 = cg * CG + p * G
        pltpu.sync_copy(ht_hbm.at[pl.ds(c0, G), b], h_v)

        def sbody(i, c2):
            iv = idx_v[pl.ds(i * L, L)]
            for g in range(G):
                hv = h_v[g, pl.ds(i * L, L)]
                plsc.addupdate_scatter(grid_v, [iv + g * V], hv)
            return c2

        lax.fori_loop(0, N // L, sbody, 0)

        for g in range(G):
            def rbody(j, acc):
                sl = pl.ds(g * V + j * L, L)
                val = grid_v[sl]
                grid_v[sl] = jnp.zeros((L,), jnp.float32)
                return jnp.maximum(acc, val)

            acc = lax.fori_loop(0, V // L, rbody, jnp.zeros((L,), jnp.float32))
            out_v[p * G + g, :] = acc
        return carry

    lax.fori_loop(0, NPASS, pass_body, 0)
    pltpu.sync_copy(out_v, out_hbm.at[b, pl.ds(cg * CG, CG)])


@functools.lru_cache(maxsize=None)
def _sc_scatter_max():
    mesh = plsc.VectorSubcoreMesh(core_axis_name="c", subcore_axis_name="s")
    return pl.kernel(
        _sc_body,
        mesh=mesh,
        compiler_params=pltpu.CompilerParams(needs_layout_passes=False),
        out_type=jax.ShapeDtypeStruct((B, C, L), jnp.float32),
        scratch_types=[
            pltpu.VMEM((N,), jnp.int32),
            pltpu.VMEM((G, N), jnp.float32),
            pltpu.VMEM((G * V,), jnp.float32),
            pltpu.VMEM((CG, L), jnp.float32),
        ],
    )


def _gn_leaky(x, gamma, beta, groups):
    cx = x.shape[1]
    gsz = cx // groups
    ci = lax.broadcasted_iota(jnp.int32, (cx, groups), 0) // gsz
    gi = lax.broadcasted_iota(jnp.int32, (cx, groups), 1)
    p = (ci == gi).astype(jnp.float32)
    cit = lax.broadcasted_iota(jnp.int32, (groups, cx), 1) // gsz
    git = lax.broadcasted_iota(jnp.int32, (groups, cx), 0)
    pt = (git == cit).astype(jnp.float32)
    mu = jnp.dot(x, p, preferred_element_type=jnp.float32) / gsz
    m2 = jnp.dot(x * x, p, preferred_element_type=jnp.float32) / gsz
    var = m2 - mu * mu
    muf = jnp.dot(mu, pt, preferred_element_type=jnp.float32)
    varf = jnp.dot(var, pt, preferred_element_type=jnp.float32)
    xn = (x - muf) * lax.rsqrt(varf + 1e-5)
    y = xn * gamma + beta
    return jnp.where(y >= 0, y, 0.01 * y)


def _head_body(sf_ref, lw0_ref, lb0_ref, lw1_ref, lb1_ref, lw2_ref, lb2_ref,
               g1g_ref, g1b_ref, g2g_ref, g2b_ref, out_ref):
    sf = jnp.max(sf_ref[...], axis=-1)
    x = jnp.dot(sf, lw0_ref[...], preferred_element_type=jnp.float32) + lb0_ref[...]
    x = _gn_leaky(x, g1g_ref[...], g1b_ref[...], 16)
    x = jnp.dot(x, lw1_ref[...], preferred_element_type=jnp.float32) + lb1_ref[...]
    x = _gn_leaky(x, g2g_ref[...], g2b_ref[...], 16)
    out_ref[...] = (
        jnp.dot(x, lw2_ref[...], preferred_element_type=jnp.float32)
        + lb2_ref[...])


_head_call = pl.pallas_call(
    _head_body,
    out_shape=jax.ShapeDtypeStruct((B, 2048), jnp.float32),
)


def kernel(scene_pc, W0, b0, W1, b1, W2, b2, LW0, Lb0, LW1, Lb1, LW2, Lb2,
           g1_g, g1_b, g2_g, g2_b):
    xt = scene_pc.reshape(PTS, 4).T
    ht, flat = _mlp_call(
        xt, W0, b0.reshape(64, 1), W1, b1.reshape(128, 1), W2,
        b2.reshape(256, 1))
    ht3 = ht.reshape(C, B, N)
    flat2 = flat.reshape(B, N)
    sf16 = _sc_scatter_max()(ht3, flat2)
    out = _head_call(sf16, LW0.T, Lb0, LW1.T, Lb1, LW2.T, Lb2,
                     g1_g, g1_b, g2_g, g2_b)
    return out


# speedup vs baseline: 2.3575x; 2.3575x over previous
import functools

import jax
import jax.numpy as jnp
from jax import lax
from jax.experimental import pallas as pl
from jax.experimental.pallas import tpu as pltpu
from jax.experimental.pallas import tpu_sc as plsc

B, N, V_SIDE = 4, 16384, 32
V = V_SIDE ** 3
C = 256
PTS = B * N
PBLK = 2048

NC, NS, L = 2, 16, 16
NW = NC * NS
CG = C // (NW // B)
G = 2
NPASS = CG // G


def _mlp_body(xt_ref, w0_ref, b0_ref, w1_ref, b1_ref, w2_ref, b2_ref,
              ht_ref, flat_ref):
    xt = xt_ref[...]
    h = jnp.maximum(
        jnp.dot(w0_ref[...], xt, preferred_element_type=jnp.float32)
        + b0_ref[...], 0.0)
    h = jnp.maximum(
        jnp.dot(w1_ref[...], h, preferred_element_type=jnp.float32)
        + b1_ref[...], 0.0)
    h = jnp.maximum(
        jnp.dot(w2_ref[...], h, preferred_element_type=jnp.float32)
        + b2_ref[...], 0.0)
    ht_ref[...] = h
    vox = jnp.floor(xt * 32.0).astype(jnp.int32)
    flat = vox[0:1] * 1024 + vox[1:2] * 32 + vox[2:3]
    flat_ref[...] = flat.reshape(1, 1, PBLK)


_mlp_call = pl.pallas_call(
    _mlp_body,
    grid=(PTS // PBLK,),
    in_specs=[
        pl.BlockSpec((4, PBLK), lambda i: (0, i)),
        pl.BlockSpec((64, 4), lambda i: (0, 0)),
        pl.BlockSpec((64, 1), lambda i: (0, 0)),
        pl.BlockSpec((128, 64), lambda i: (0, 0)),
        pl.BlockSpec((128, 1), lambda i: (0, 0)),
        pl.BlockSpec((256, 128), lambda i: (0, 0)),
        pl.BlockSpec((256, 1), lambda i: (0, 0)),
    ],
    out_specs=[
        pl.BlockSpec((C, PBLK), lambda i: (0, i)),
        pl.BlockSpec((1, 1, PBLK), lambda i: (i, 0, 0)),
    ],
    out_shape=[
        jax.ShapeDtypeStruct((C, PTS), jnp.float32),
        jax.ShapeDtypeStruct((PTS // PBLK, 1, PBLK), jnp.int32),
    ],
)


def _sc_body(ht_hbm, flat_hbm, out_hbm, idx_v, h_v, grid_v, out_v):
    w = lax.axis_index("s") * NC + lax.axis_index("c")
    b = w // (NW // B)
    cg = w % (NW // B)
    pltpu.sync_copy(flat_hbm.at[b], idx_v)

    zero = jnp.zeros((L,), jnp.float32)

    @plsc.parallel_loop(0, G * V // L, unroll=8)
    def _(i):
        grid_v[pl.ds(i * L, L)] = zero

    def pass_body(p, carry):
        c0 = cg * CG + p * G
        pltpu.sync_copy(ht_hbm.at[pl.ds(c0, G), b], h_v)

        @plsc.parallel_loop(0, N // L, unroll=8)
        def _(i):
            iv = idx_v[pl.ds(i * L, L)]
            for g in range(G):
                hv = h_v[g, pl.ds(i * L, L)]
                plsc.addupdate_scatter(grid_v, [iv + g * V], hv)

        for g in range(G):
            @plsc.parallel_loop(0, V // (4 * L), unroll=4,
                                carry=(zero, zero, zero, zero))
            def accs(j, accs):
                base = g * V + j * (4 * L)
                out = []
                for k in range(4):
                    sl = pl.ds(base + k * L, L)
                    out.append(jnp.maximum(accs[k], grid_v[sl]))
                    grid_v[sl] = zero
                return tuple(out)

            acc = jnp.maximum(jnp.maximum(accs[0], accs[1]),
                              jnp.maximum(accs[2], accs[3]))
            out_v[p * G + g, :] = acc
        return carry

    lax.fori_loop(0, NPASS, pass_body, 0)
    pltpu.sync_copy(out_v, out_hbm.at[b, pl.ds(cg * CG, CG)])


@functools.lru_cache(maxsize=None)
def _sc_scatter_max():
    mesh = plsc.VectorSubcoreMesh(core_axis_name="c", subcore_axis_name="s")
    return pl.kernel(
        _sc_body,
        mesh=mesh,
        compiler_params=pltpu.CompilerParams(needs_layout_passes=False),
        out_type=jax.ShapeDtypeStruct((B, C, L), jnp.float32),
        scratch_types=[
            pltpu.VMEM((N,), jnp.int32),
            pltpu.VMEM((G, N), jnp.float32),
            pltpu.VMEM((G * V,), jnp.float32),
            pltpu.VMEM((CG, L), jnp.float32),
        ],
    )


def _gn_leaky(x, gamma, beta, groups):
    cx = x.shape[1]
    gsz = cx // groups
    ci = lax.broadcasted_iota(jnp.int32, (cx, groups), 0) // gsz
    gi = lax.broadcasted_iota(jnp.int32, (cx, groups), 1)
    p = (ci == gi).astype(jnp.float32)
    cit = lax.broadcasted_iota(jnp.int32, (groups, cx), 1) // gsz
    git = lax.broadcasted_iota(jnp.int32, (groups, cx), 0)
    pt = (git == cit).astype(jnp.float32)
    mu = jnp.dot(x, p, preferred_element_type=jnp.float32) / gsz
    m2 = jnp.dot(x * x, p, preferred_element_type=jnp.float32) / gsz
    var = m2 - mu * mu
    muf = jnp.dot(mu, pt, preferred_element_type=jnp.float32)
    varf = jnp.dot(var, pt, preferred_element_type=jnp.float32)
    xn = (x - muf) * lax.rsqrt(varf + 1e-5)
    y = xn * gamma + beta
    return jnp.where(y >= 0, y, 0.01 * y)


def _head_body(sf_ref, lw0_ref, lb0_ref, lw1_ref, lb1_ref, lw2_ref, lb2_ref,
               g1g_ref, g1b_ref, g2g_ref, g2b_ref, out_ref):
    sf = jnp.max(sf_ref[...], axis=-1)
    x = jnp.dot(sf, lw0_ref[...], preferred_element_type=jnp.float32) + lb0_ref[...]
    x = _gn_leaky(x, g1g_ref[...], g1b_ref[...], 16)
    x = jnp.dot(x, lw1_ref[...], preferred_element_type=jnp.float32) + lb1_ref[...]
    x = _gn_leaky(x, g2g_ref[...], g2b_ref[...], 16)
    out_ref[...] = (
        jnp.dot(x, lw2_ref[...], preferred_element_type=jnp.float32)
        + lb2_ref[...])


_head_call = pl.pallas_call(
    _head_body,
    out_shape=jax.ShapeDtypeStruct((B, 2048), jnp.float32),
)


def kernel(scene_pc, W0, b0, W1, b1, W2, b2, LW0, Lb0, LW1, Lb1, LW2, Lb2,
           g1_g, g1_b, g2_g, g2_b):
    xt = scene_pc.reshape(PTS, 4).T
    ht, flat = _mlp_call(
        xt, W0, b0.reshape(64, 1), W1, b1.reshape(128, 1), W2,
        b2.reshape(256, 1))
    ht3 = ht.reshape(C, B, N)
    flat2 = flat.reshape(B, N)
    sf16 = _sc_scatter_max()(ht3, flat2)
    out = _head_call(sf16, LW0.T, Lb0, LW1.T, Lb1, LW2.T, Lb2,
                     g1_g, g1_b, g2_g, g2_b)
    return out
